# Initial kernel scaffold; baseline (speedup 1.0000x reference)
import functools

import jax
import jax.numpy as jnp
from jax import lax
from jax.experimental import pallas as pl
from jax.experimental.pallas import tpu as pltpu
from jax.experimental.pallas import tpu_sc as plsc

VOCAB = 100000
D = 128
B = 4
T_TEXT = 2048
T_ADD = 512
T_OUT = T_TEXT + T_ADD
N_ROWS = B * T_OUT
NW = 32
ROWS_PER_W = N_ROWS // NW
CHUNK = 64
N_CHUNKS = ROWS_PER_W // CHUNK

_mesh = plsc.VectorSubcoreMesh(core_axis_name="c", subcore_axis_name="s")


@functools.partial(
    pl.kernel,
    out_type=[
        jax.ShapeDtypeStruct((N_ROWS, D), jnp.float32),
        jax.ShapeDtypeStruct((N_ROWS,), jnp.int32),
    ],
    mesh=_mesh,
    scratch_types=[
        pltpu.VMEM((N_CHUNKS, CHUNK), jnp.int32),
        pltpu.VMEM((CHUNK, D), jnp.float32),
        pltpu.VMEM((T_TEXT,), jnp.int32),
        pltpu.SemaphoreType.DMA,
    ],
)
def _gather_concat(ids_hbm, am_hbm, addm_hbm, w_hbm,
                   out_emb, out_mask, idx_v, rows_v, mbuf, sem):
    wid = lax.axis_index("s") * 2 + lax.axis_index("c")
    base = wid * ROWS_PER_W

    pltpu.sync_copy(ids_hbm.at[pl.ds(wid * N_CHUNKS, N_CHUNKS)], idx_v)

    for j in range(N_CHUNKS):
        pltpu.async_copy(w_hbm.at[idx_v.at[j]], rows_v, sem).wait()
        pltpu.sync_copy(rows_v, out_emb.at[pl.ds(base + j * CHUNK, CHUNK)])

    @pl.when(wid < B)
    def _():
        pltpu.sync_copy(am_hbm.at[pl.ds(wid * T_TEXT, T_TEXT)], mbuf)
        pltpu.sync_copy(mbuf, out_mask.at[pl.ds(wid * T_OUT, T_TEXT)])
        pltpu.sync_copy(addm_hbm.at[pl.ds(wid * T_ADD, T_ADD)],
                        mbuf.at[pl.ds(0, T_ADD)])
        pltpu.sync_copy(mbuf.at[pl.ds(0, T_ADD)],
                        out_mask.at[pl.ds(wid * T_OUT + T_TEXT, T_ADD)])


def kernel(input_ids, attention_mask, add_ids, add_mask, W):
    ids = jnp.concatenate([input_ids, add_ids], axis=1).reshape(NW * N_CHUNKS,
                                                                CHUNK)
    emb, mask = _gather_concat(ids, attention_mask.reshape(-1),
                               add_mask.reshape(-1), W)
    return emb.reshape(B, T_OUT, D), mask.reshape(B, T_OUT)


# speedup vs baseline: 1.1881x; 1.1881x over previous
import functools

import jax
import jax.numpy as jnp
from jax import lax
from jax.experimental import pallas as pl
from jax.experimental.pallas import tpu as pltpu
from jax.experimental.pallas import tpu_sc as plsc

VOCAB = 100000
D = 128
B = 4
T_TEXT = 2048
T_ADD = 512
T_OUT = T_TEXT + T_ADD
N_ROWS = B * T_OUT
NW = 32
ROWS_PER_W = N_ROWS // NW
CHUNK = 64
N_CHUNKS = ROWS_PER_W // CHUNK

_mesh = plsc.VectorSubcoreMesh(core_axis_name="c", subcore_axis_name="s")


@functools.partial(
    pl.kernel,
    out_type=[
        jax.ShapeDtypeStruct((N_ROWS, D), jnp.float32),
        jax.ShapeDtypeStruct((N_ROWS,), jnp.int32),
    ],
    mesh=_mesh,
    scratch_types=[
        pltpu.VMEM((ROWS_PER_W,), jnp.int32),
        pltpu.VMEM((CHUNK, D), jnp.float32),
        pltpu.VMEM((T_TEXT,), jnp.int32),
        pltpu.SemaphoreType.DMA,
    ],
)
def _gather_concat(ids_hbm, am_hbm, addm_hbm, w_hbm,
                   out_emb, out_mask, idx_v, rows_v, mbuf, sem):
    wid = lax.axis_index("s") * 2 + lax.axis_index("c")
    base = wid * ROWS_PER_W

    pltpu.sync_copy(ids_hbm.at[pl.ds(base, ROWS_PER_W)], idx_v)

    for j in range(N_CHUNKS):
        pltpu.async_copy(w_hbm.at[idx_v.at[pl.ds(j * CHUNK, CHUNK)]],
                         rows_v, sem).wait()
        pltpu.sync_copy(rows_v, out_emb.at[pl.ds(base + j * CHUNK, CHUNK)])

    @pl.when(wid < B)
    def _():
        pltpu.sync_copy(am_hbm.at[pl.ds(wid * T_TEXT, T_TEXT)], mbuf)
        pltpu.sync_copy(mbuf, out_mask.at[pl.ds(wid * T_OUT, T_TEXT)])
        pltpu.sync_copy(addm_hbm.at[pl.ds(wid * T_ADD, T_ADD)],
                        mbuf.at[pl.ds(0, T_ADD)])
        pltpu.sync_copy(mbuf.at[pl.ds(0, T_ADD)],
                        out_mask.at[pl.ds(wid * T_OUT + T_TEXT, T_ADD)])


def kernel(input_ids, attention_mask, add_ids, add_mask, W):
    ids = jnp.concatenate([input_ids, add_ids], axis=1).reshape(-1)
    emb, mask = _gather_concat(ids, attention_mask.reshape(-1),
                               add_mask.reshape(-1), W)
    return emb.reshape(B, T_OUT, D), mask.reshape(B, T_OUT)


# speedup vs baseline: 1.3925x; 1.1721x over previous
import functools

import jax
import jax.numpy as jnp
from jax import lax
from jax.experimental import pallas as pl
from jax.experimental.pallas import tpu as pltpu
from jax.experimental.pallas import tpu_sc as plsc

VOCAB = 100000
D = 128
B = 4
T_TEXT = 2048
T_ADD = 512
T_OUT = T_TEXT + T_ADD
N_ROWS = B * T_OUT
NW = 32
ROWS_PER_W = N_ROWS // NW
CHUNK = 128
CHUNKS = [(0, CHUNK), (CHUNK, CHUNK), (2 * CHUNK, ROWS_PER_W - 2 * CHUNK)]

_mesh = plsc.VectorSubcoreMesh(core_axis_name="c", subcore_axis_name="s")


@functools.partial(
    pl.kernel,
    out_type=[
        jax.ShapeDtypeStruct((N_ROWS, D), jnp.float32),
        jax.ShapeDtypeStruct((N_ROWS,), jnp.int32),
    ],
    mesh=_mesh,
    scratch_types=[
        pltpu.VMEM((ROWS_PER_W,), jnp.int32),
        pltpu.VMEM((ROWS_PER_W, D), jnp.float32),
        pltpu.VMEM((T_TEXT,), jnp.int32),
        pltpu.VMEM((T_ADD,), jnp.int32),
        pltpu.SemaphoreType.DMA,
        pltpu.SemaphoreType.DMA,
    ],
)
def _gather_concat(ids_hbm, am_hbm, addm_hbm, w_hbm,
                   out_emb, out_mask, idx_v, rows_v, mbuf, abuf, sem, msem):
    wid = lax.axis_index("s") * 2 + lax.axis_index("c")
    base = wid * ROWS_PER_W
    is_mask_worker = wid < B

    pltpu.sync_copy(ids_hbm.at[pl.ds(base, ROWS_PER_W)], idx_v)

    @pl.when(is_mask_worker)
    def _():
        pltpu.async_copy(am_hbm.at[pl.ds(wid * T_TEXT, T_TEXT)], mbuf, msem)
        pltpu.async_copy(addm_hbm.at[pl.ds(wid * T_ADD, T_ADD)], abuf, msem)

    gathers = [
        pltpu.async_copy(w_hbm.at[idx_v.at[pl.ds(off, n)]],
                         rows_v.at[pl.ds(off, n)], sem)
        for off, n in CHUNKS
    ]

    @pl.when(is_mask_worker)
    def _():
        pltpu.make_async_copy(am_hbm.at[pl.ds(0, T_TEXT)], mbuf, msem).wait()
        pltpu.make_async_copy(addm_hbm.at[pl.ds(0, T_ADD)], abuf, msem).wait()
        pltpu.async_copy(mbuf, out_mask.at[pl.ds(wid * T_OUT, T_TEXT)], msem)
        pltpu.async_copy(abuf, out_mask.at[pl.ds(wid * T_OUT + T_TEXT, T_ADD)],
                         msem)

    for g in gathers:
        g.wait()

    pltpu.sync_copy(rows_v, out_emb.at[pl.ds(base, ROWS_PER_W)])

    @pl.when(is_mask_worker)
    def _():
        pltpu.make_async_copy(mbuf, out_mask.at[pl.ds(0, T_TEXT)],
                              msem).wait()
        pltpu.make_async_copy(abuf, out_mask.at[pl.ds(0, T_ADD)], msem).wait()


def kernel(input_ids, attention_mask, add_ids, add_mask, W):
    ids = jnp.concatenate([input_ids, add_ids], axis=1).reshape(-1)
    emb, mask = _gather_concat(ids, attention_mask.reshape(-1),
                               add_mask.reshape(-1), W)
    return emb.reshape(B, T_OUT, D), mask.reshape(B, T_OUT)
